# baseline (speedup 1.0000x reference)
import functools

import jax
import jax.numpy as jnp
from jax import lax
from jax.experimental import pallas as pl
from jax.experimental.pallas import tpu as pltpu
from jax.experimental.pallas import tpu_sc as plsc

_W = 16
_L = 16
_NC = 2
_NS = 16
_NW = _NC * _NS
_R = 16


def _make_sc_add(rows, d):
    rpw = rows // _NW
    nchunk = rpw // _R
    mesh = plsc.VectorSubcoreMesh(core_axis_name="c", subcore_axis_name="s")

    @functools.partial(
        pl.kernel,
        mesh=mesh,
        out_type=jax.ShapeDtypeStruct((rows, d), jnp.float32),
        scratch_types=[
            pltpu.VMEM((_W, d), jnp.float32),
            pltpu.VMEM((_R, d), jnp.float32),
            pltpu.VMEM((_R, d), jnp.float32),
            pltpu.VMEM((_R, d), jnp.float32),
            pltpu.VMEM((_R, d), jnp.float32),
            pltpu.SemaphoreType.DMA,
            pltpu.SemaphoreType.DMA,
            pltpu.SemaphoreType.DMA,
            pltpu.SemaphoreType.DMA,
        ],
    )
    def sc_add(x_hbm, t_hbm, o_hbm, tab, ib0, ib1, ob0, ob1, si0, si1, so0, so1):
        wid = lax.axis_index("s") * _NC + lax.axis_index("c")
        base = wid * rpw
        ibs, obs, sis, sos = (ib0, ib1), (ob0, ob1), (si0, si1), (so0, so1)

        def cin(g, b):
            return pltpu.make_async_copy(
                x_hbm.at[pl.ds(base + g * _R, _R)], ibs[b], sis[b])

        def cout(g, b):
            return pltpu.make_async_copy(
                obs[b], o_hbm.at[pl.ds(base + g * _R, _R)], sos[b])

        pltpu.sync_copy(t_hbm, tab)

        def compute(b):
            ib, ob = ibs[b], obs[b]

            def jbody(j, c):
                s = pl.ds(j * _L, _L)
                for k in range(_R):
                    ob[k, s] = ib[k, s] + tab[k % _W, s]
                return c

            lax.fori_loop(0, d // _L, jbody, 0)

        cin(0, 0).start()
        cin(1, 1).start()
        for g in (0, 1):
            b = g & 1
            cin(g, b).wait()
            compute(b)
            cout(g, b).start()
            cin(g + 2, b).start()

        def gbody(i, c):
            g0 = 2 * i
            for b in (0, 1):
                g = g0 + b
                cin(g, b).wait()
                cout(g - 2, b).wait()
                compute(b)
                cout(g, b).start()
                cin(g + 2, b).start()
            return c

        lax.fori_loop(1, nchunk // 2 - 1, gbody, 0)

        for g in (nchunk - 2, nchunk - 1):
            b = g & 1
            cin(g, b).wait()
            cout(g - 2, b).wait()
            compute(b)
            cout(g, b).start()
        for g in (nchunk - 2, nchunk - 1):
            cout(g, g & 1).wait()

    return sc_add


def kernel(visn_feats, table):
    B, S, D = visn_feats.shape
    rows = B * S
    x2 = visn_feats.reshape(rows, D)
    out = _make_sc_add(rows, D)(x2, table)
    return out.reshape(B, S, D)
